# Initial kernel scaffold; baseline (speedup 1.0000x reference)
import jax
import jax.numpy as jnp
from jax.experimental import pallas as pl


def kernel(objectness, box_regression, anchors):
    raise NotImplementedError("write your pallas kernel here")


# speedup vs baseline: 19.7526x; 19.7526x over previous
import functools

import jax
import jax.numpy as jnp
import numpy as np
from jax.experimental import pallas as pl
from jax.experimental.pallas import tpu as pltpu

_N, _A, _H, _W = 2, 3, 64, 64
_IMG_H, _IMG_W = 1024.0, 1024.0
_PRE_NMS = 2000
_POST_NMS = 1000
_NMS_THRESH = 0.7
_CLIP = float(np.log(1000.0 / 16.0))
_NPAD = 2048
_CHUNK = 256
_NCHUNKS = _NPAD // _CHUNK


def _decode_clip(dx, dy, dw, dh, ax1, ay1, ax2, ay2):
    widths = ax2 - ax1 + 1.0
    heights = ay2 - ay1 + 1.0
    ctr_x = ax1 + 0.5 * widths
    ctr_y = ay1 + 0.5 * heights
    dw = jnp.minimum(dw, _CLIP)
    dh = jnp.minimum(dh, _CLIP)
    pred_ctr_x = dx * widths + ctr_x
    pred_ctr_y = dy * heights + ctr_y
    pred_w = jnp.exp(dw) * widths
    pred_h = jnp.exp(dh) * heights
    x1 = pred_ctr_x - 0.5 * pred_w
    y1 = pred_ctr_y - 0.5 * pred_h
    x2 = pred_ctr_x + 0.5 * pred_w - 1.0
    y2 = pred_ctr_y + 0.5 * pred_h - 1.0
    x1 = jnp.clip(x1, 0.0, _IMG_W - 1.0)
    y1 = jnp.clip(y1, 0.0, _IMG_H - 1.0)
    x2 = jnp.clip(x2, 0.0, _IMG_W - 1.0)
    y2 = jnp.clip(y2, 0.0, _IMG_H - 1.0)
    return x1, y1, x2, y2


def _nms_body(breg_r_ref, anch_r_ref, breg_c_ref, anch_c_ref,
              boxes_out_ref, keep_out_ref, iou_scr):
    br = breg_r_ref[0]
    ar = anch_r_ref[0]
    bc = breg_c_ref[0]
    ac = anch_c_ref[0]

    x1r, y1r, x2r, y2r = _decode_clip(
        br[0:1, :], br[1:2, :], br[2:3, :], br[3:4, :],
        ar[0:1, :], ar[1:2, :], ar[2:3, :], ar[3:4, :])
    x1c, y1c, x2c, y2c = _decode_clip(
        bc[:, 0:1], bc[:, 1:2], bc[:, 2:3], bc[:, 3:4],
        ac[:, 0:1], ac[:, 1:2], ac[:, 2:3], ac[:, 3:4])

    area_r = (x2r - x1r + 1.0) * (y2r - y1r + 1.0)
    area_c = (x2c - x1c + 1.0) * (y2c - y1c + 1.0)

    iota = jax.lax.broadcasted_iota(jnp.int32, (1, _NPAD), 1)
    ws = x2r - x1r + 1.0
    hs = y2r - y1r + 1.0
    valid = (ws >= 0.0) & (hs >= 0.0) & (iota < _PRE_NMS)
    keep = valid.astype(jnp.float32)

    iota256 = jax.lax.broadcasted_iota(jnp.int32, (1, _CHUNK), 1)

    for c in range(_NCHUNKS):
        s, e = c * _CHUNK, (c + 1) * _CHUNK
        ltx = jnp.maximum(x1c[s:e, :], x1r)
        lty = jnp.maximum(y1c[s:e, :], y1r)
        rbx = jnp.minimum(x2c[s:e, :], x2r)
        rby = jnp.minimum(y2c[s:e, :], y2r)
        w = jnp.maximum(rbx - ltx + 1.0, 0.0)
        h = jnp.maximum(rby - lty + 1.0, 0.0)
        inter = w * h
        iou = inter / (area_c[s:e, :] + area_r - inter)
        sup = (iou > _NMS_THRESH).astype(jnp.float32)

        iou_scr[...] = sup[:, s:e]
        k0 = keep[:, s:e]

        def body(i, k):
            ki = jnp.sum(k * (iota256 == i).astype(jnp.float32))
            row = iou_scr[pl.ds(i, 1), :]
            after = (iota256 > i).astype(jnp.float32)
            return k * (1.0 - ki * row * after)

        k_fin = jax.lax.fori_loop(0, _CHUNK, body, k0)

        cnt = jnp.dot(k_fin, sup, preferred_element_type=jnp.float32)
        later = ((cnt > 0.5) & (iota >= e)).astype(jnp.float32)
        keep = keep * (1.0 - later)
        parts = []
        if s > 0:
            parts.append(keep[:, :s])
        parts.append(k_fin)
        if e < _NPAD:
            parts.append(keep[:, e:])
        keep = jnp.concatenate(parts, axis=1) if len(parts) > 1 else k_fin

    boxes_out_ref[0] = jnp.concatenate([x1r, y1r, x2r, y2r], axis=0)
    keep_out_ref[0] = keep


@jax.jit
def kernel(objectness, box_regression, anchors):
    n = objectness.shape[0]
    obj = objectness.transpose(0, 2, 3, 1).reshape(n, -1)
    obj = jax.nn.sigmoid(obj)
    scores, topk_idx = jax.lax.top_k(obj, _PRE_NMS)
    breg = box_regression.reshape(n, _A, 4, _H, _W)
    breg = breg.transpose(0, 3, 4, 1, 2).reshape(n, -1, 4)
    breg = jnp.take_along_axis(breg, topk_idx[:, :, None], axis=1)
    anch = jnp.take_along_axis(anchors, topk_idx[:, :, None], axis=1)

    pad = _NPAD - _PRE_NMS
    breg_c = jnp.pad(breg, ((0, 0), (0, pad), (0, 0)))
    anch_c = jnp.pad(anch, ((0, 0), (0, pad), (0, 0)))
    breg_r = breg_c.transpose(0, 2, 1)
    anch_r = anch_c.transpose(0, 2, 1)

    boxes_t, keep = pl.pallas_call(
        _nms_body,
        grid=(n,),
        in_specs=[
            pl.BlockSpec((1, 4, _NPAD), lambda i: (i, 0, 0)),
            pl.BlockSpec((1, 4, _NPAD), lambda i: (i, 0, 0)),
            pl.BlockSpec((1, _NPAD, 4), lambda i: (i, 0, 0)),
            pl.BlockSpec((1, _NPAD, 4), lambda i: (i, 0, 0)),
        ],
        out_specs=[
            pl.BlockSpec((1, 4, _NPAD), lambda i: (i, 0, 0)),
            pl.BlockSpec((1, 1, _NPAD), lambda i: (i, 0, 0)),
        ],
        out_shape=[
            jax.ShapeDtypeStruct((n, 4, _NPAD), jnp.float32),
            jax.ShapeDtypeStruct((n, 1, _NPAD), jnp.float32),
        ],
        scratch_shapes=[pltpu.VMEM((_CHUNK, _CHUNK), jnp.float32)],
    )(breg_r, anch_r, breg_c, anch_c)

    proposals = boxes_t.transpose(0, 2, 1)[:, :_PRE_NMS]
    keep_b = keep[:, 0, :_PRE_NMS] > 0.5
    masked = jnp.where(keep_b, scores, -1e10)
    topv, topi = jax.lax.top_k(masked, _POST_NMS)
    out_boxes = jnp.take_along_axis(proposals, topi[:, :, None], axis=1)
    return jnp.concatenate([out_boxes, topv[:, :, None]], axis=-1)


# speedup vs baseline: 19.8572x; 1.0053x over previous
import functools

import jax
import jax.numpy as jnp
import numpy as np
from jax.experimental import pallas as pl
from jax.experimental.pallas import tpu as pltpu

_N, _A, _H, _W = 2, 3, 64, 64
_IMG_H, _IMG_W = 1024.0, 1024.0
_PRE_NMS = 2000
_POST_NMS = 1000
_NMS_THRESH = 0.7
_CLIP = float(np.log(1000.0 / 16.0))
_NPAD = 2048
_CHUNK = 256
_NCHUNKS = _NPAD // _CHUNK
_POST_PAD = 1024


def _decode_clip(dx, dy, dw, dh, ax1, ay1, ax2, ay2):
    widths = ax2 - ax1 + 1.0
    heights = ay2 - ay1 + 1.0
    ctr_x = ax1 + 0.5 * widths
    ctr_y = ay1 + 0.5 * heights
    dw = jnp.minimum(dw, _CLIP)
    dh = jnp.minimum(dh, _CLIP)
    pred_ctr_x = dx * widths + ctr_x
    pred_ctr_y = dy * heights + ctr_y
    pred_w = jnp.exp(dw) * widths
    pred_h = jnp.exp(dh) * heights
    x1 = pred_ctr_x - 0.5 * pred_w
    y1 = pred_ctr_y - 0.5 * pred_h
    x2 = pred_ctr_x + 0.5 * pred_w - 1.0
    y2 = pred_ctr_y + 0.5 * pred_h - 1.0
    x1 = jnp.clip(x1, 0.0, _IMG_W - 1.0)
    y1 = jnp.clip(y1, 0.0, _IMG_H - 1.0)
    x2 = jnp.clip(x2, 0.0, _IMG_W - 1.0)
    y2 = jnp.clip(y2, 0.0, _IMG_H - 1.0)
    return x1, y1, x2, y2


def _nms_body(breg_r_ref, anch_r_ref, breg_c_ref, anch_c_ref, scores_ref,
              out_ref, iou_scr):
    br = breg_r_ref[0]
    ar = anch_r_ref[0]
    bc = breg_c_ref[0]
    ac = anch_c_ref[0]

    x1r, y1r, x2r, y2r = _decode_clip(
        br[0:1, :], br[1:2, :], br[2:3, :], br[3:4, :],
        ar[0:1, :], ar[1:2, :], ar[2:3, :], ar[3:4, :])
    x1c, y1c, x2c, y2c = _decode_clip(
        bc[:, 0:1], bc[:, 1:2], bc[:, 2:3], bc[:, 3:4],
        ac[:, 0:1], ac[:, 1:2], ac[:, 2:3], ac[:, 3:4])

    area_r = (x2r - x1r + 1.0) * (y2r - y1r + 1.0)
    area_c = (x2c - x1c + 1.0) * (y2c - y1c + 1.0)

    iota = jax.lax.broadcasted_iota(jnp.int32, (1, _NPAD), 1)
    ws = x2r - x1r + 1.0
    hs = y2r - y1r + 1.0
    valid = (ws >= 0.0) & (hs >= 0.0) & (iota < _PRE_NMS)
    keep = valid.astype(jnp.float32)

    iota256 = jax.lax.broadcasted_iota(jnp.int32, (1, _CHUNK), 1)

    for c in range(_NCHUNKS):
        s, e = c * _CHUNK, (c + 1) * _CHUNK
        ltx = jnp.maximum(x1c[s:e, :], x1r)
        lty = jnp.maximum(y1c[s:e, :], y1r)
        rbx = jnp.minimum(x2c[s:e, :], x2r)
        rby = jnp.minimum(y2c[s:e, :], y2r)
        w = jnp.maximum(rbx - ltx + 1.0, 0.0)
        h = jnp.maximum(rby - lty + 1.0, 0.0)
        inter = w * h
        iou = inter / (area_c[s:e, :] + area_r - inter)
        sup = (iou > _NMS_THRESH).astype(jnp.float32)

        iou_scr[...] = sup[:, s:e]
        k0 = keep[:, s:e]

        def body(i, k):
            ki = jnp.sum(k * (iota256 == i).astype(jnp.float32))
            row = iou_scr[pl.ds(i, 1), :]
            after = (iota256 > i).astype(jnp.float32)
            return k * (1.0 - ki * row * after)

        k_fin = jax.lax.fori_loop(0, _CHUNK, body, k0)

        cnt = jnp.dot(k_fin, sup, preferred_element_type=jnp.float32)
        later = ((cnt > 0.5) & (iota >= e)).astype(jnp.float32)
        keep = keep * (1.0 - later)
        parts = []
        if s > 0:
            parts.append(keep[:, :s])
        parts.append(k_fin)
        if e < _NPAD:
            parts.append(keep[:, e:])
        keep = jnp.concatenate(parts, axis=1) if len(parts) > 1 else k_fin

    score_row = scores_ref[0]
    real_f = (iota < _PRE_NMS).astype(jnp.float32)
    s_row = (1.0 - keep) * real_f
    tri = (jax.lax.broadcasted_iota(jnp.int32, (_CHUNK, _CHUNK), 0)
           <= jax.lax.broadcasted_iota(jnp.int32, (_CHUNK, _CHUNK), 1)
           ).astype(jnp.float32)
    kpre = jnp.float32(0.0)
    spre = jnp.float32(0.0)
    kparts, sparts = [], []
    for c in range(_NCHUNKS):
        s0, e0 = c * _CHUNK, (c + 1) * _CHUNK
        kc = keep[:, s0:e0]
        sc = s_row[:, s0:e0]
        kparts.append(kpre + jnp.dot(kc, tri, preferred_element_type=jnp.float32))
        sparts.append(spre + jnp.dot(sc, tri, preferred_element_type=jnp.float32))
        kpre = kpre + jnp.sum(kc)
        spre = spre + jnp.sum(sc)
    rank = jnp.concatenate(kparts, axis=1)
    srank = jnp.concatenate(sparts, axis=1)
    num_keep = kpre
    dest = keep * (rank - 1.0) + (1.0 - keep) * (num_keep + srank - 1.0)
    dest_i = dest.astype(jnp.int32)
    masked = keep * score_row + (1.0 - keep) * (-1e10)

    comps = (x1r, y1r, x2r, y2r, masked)
    for pc in range(_POST_PAD // _CHUNK):
        p0 = pc * _CHUNK
        pio = p0 + jax.lax.broadcasted_iota(jnp.int32, (_CHUNK, _NPAD), 0)
        match = dest_i == pio
        for ci, comp in enumerate(comps):
            sel = jnp.where(match, comp, -3.4e38)
            val = jnp.max(sel, axis=1, keepdims=True)
            out_ref[0, p0:p0 + _CHUNK, ci:ci + 1] = val


@jax.jit
def kernel(objectness, box_regression, anchors):
    n = objectness.shape[0]
    obj = objectness.transpose(0, 2, 3, 1).reshape(n, -1)
    obj = jax.nn.sigmoid(obj)
    scores, topk_idx = jax.lax.top_k(obj, _PRE_NMS)
    breg = box_regression.reshape(n, _A, 4, _H, _W)
    breg = breg.transpose(0, 3, 4, 1, 2).reshape(n, -1, 4)
    breg = jnp.take_along_axis(breg, topk_idx[:, :, None], axis=1)
    anch = jnp.take_along_axis(anchors, topk_idx[:, :, None], axis=1)

    pad = _NPAD - _PRE_NMS
    breg_c = jnp.pad(breg, ((0, 0), (0, pad), (0, 0)))
    anch_c = jnp.pad(anch, ((0, 0), (0, pad), (0, 0)))
    breg_r = breg_c.transpose(0, 2, 1)
    anch_r = anch_c.transpose(0, 2, 1)
    scores_p = jnp.pad(scores, ((0, 0), (0, pad)))[:, None, :]

    out = pl.pallas_call(
        _nms_body,
        grid=(n,),
        in_specs=[
            pl.BlockSpec((1, 4, _NPAD), lambda i: (i, 0, 0)),
            pl.BlockSpec((1, 4, _NPAD), lambda i: (i, 0, 0)),
            pl.BlockSpec((1, _NPAD, 4), lambda i: (i, 0, 0)),
            pl.BlockSpec((1, _NPAD, 4), lambda i: (i, 0, 0)),
            pl.BlockSpec((1, 1, _NPAD), lambda i: (i, 0, 0)),
        ],
        out_specs=pl.BlockSpec((1, _POST_PAD, 8), lambda i: (i, 0, 0)),
        out_shape=jax.ShapeDtypeStruct((n, _POST_PAD, 8), jnp.float32),
        scratch_shapes=[pltpu.VMEM((_CHUNK, _CHUNK), jnp.float32)],
    )(breg_r, anch_r, breg_c, anch_c, scores_p)

    return out[:, :_POST_NMS, :5]
